# baseline (speedup 1.0000x reference)
import jax
import jax.numpy as jnp
from jax.experimental import pallas as pl
from jax.experimental.pallas import tpu as pltpu

_N = 2048
_E = 32768
_M = 8
_NPTS = _M + 1
_CB = 256


def _coeffs():
    tmpN = jnp.arange(_NPTS, dtype=jnp.float32)
    num = jnp.cos(jnp.pi * (tmpN + 0.5) / _NPTS)
    kx = jnp.exp(-1.0 * (1.0 * num + 1.0)).reshape(-1, 1)
    rows = []
    for o in range(_M + 1):
        rows.append((2.0 / _NPTS) * (jnp.cos(jnp.pi * o * (tmpN + 0.5) / _NPTS).reshape(1, -1) @ kx))
    return jnp.concatenate(rows, axis=0)[:, 0]


def _cheb_body(c_ref, f_ref, t1_ref, r_ref):
    j = pl.program_id(0)
    F = f_ref[...]
    t1 = t1_ref[...]
    row = jax.lax.broadcasted_iota(jnp.int32, (_N, _CB), 0)
    col = jax.lax.broadcasted_iota(jnp.int32, (_N, _CB), 1) + j * _CB
    eye = jnp.where(row == col, jnp.float32(1.0), jnp.float32(0.0))
    told = eye
    tcur = t1
    r = eye * (0.5 * c_ref[0]) + t1 * c_ref[1]
    for k in range(2, _M + 1):
        tnew = jnp.dot(F, tcur, preferred_element_type=jnp.float32) - told
        r = r + tnew * c_ref[k]
        told = tcur
        tcur = tnew
    r_ref[...] = r


def _cheb_recursion(c, F, T1):
    return pl.pallas_call(
        _cheb_body,
        grid=(_N // _CB,),
        in_specs=[
            pl.BlockSpec(memory_space=pltpu.SMEM),
            pl.BlockSpec((_N, _N), lambda j: (0, 0)),
            pl.BlockSpec((_N, _CB), lambda j: (0, j)),
        ],
        out_specs=pl.BlockSpec((_N, _CB), lambda j: (0, j)),
        out_shape=jax.ShapeDtypeStruct((_N, _N), jnp.float32),
    )(c, F, T1)


@jax.jit
def _run(edge_index, edge_weight):
    c = _coeffs()
    eye = jnp.eye(_N, dtype=jnp.float32)
    A_set = jnp.zeros((_N, _N), jnp.float32).at[edge_index[0], edge_index[1]].set(edge_weight)
    A_sum = jnp.zeros((_N, _N), jnp.float32).at[edge_index[0], edge_index[1]].add(edge_weight)
    T1 = A_set - eye
    F = 2.0 * (A_sum - eye)
    return _cheb_recursion(c, F, T1)


def kernel(edge_index, edge_weight):
    return _run(edge_index, edge_weight)


# speedup vs baseline: 1.0304x; 1.0304x over previous
import jax
import jax.numpy as jnp
import numpy as np
from jax.experimental import pallas as pl
from jax.experimental.pallas import tpu as pltpu

_N = 2048
_E = 32768
_M = 8
_NPTS = _M + 1
_CB = 256
_W = 256


def _coeffs():
    tmpN = jnp.arange(_NPTS, dtype=jnp.float32)
    num = jnp.cos(jnp.pi * (tmpN + 0.5) / _NPTS)
    kx = jnp.exp(-1.0 * (1.0 * num + 1.0)).reshape(-1, 1)
    rows = []
    for o in range(_M + 1):
        rows.append((2.0 / _NPTS) * (jnp.cos(jnp.pi * o * (tmpN + 0.5) / _NPTS).reshape(1, -1) @ kx))
    return jnp.concatenate(rows, axis=0)[:, 0]


def _s_mono_matrix():
    rows = np.zeros((9, 9))
    rows[0, 0] = 1.0
    rows[1, 1] = 0.5
    for k in range(2, 9):
        rows[k, 1:] = rows[k - 1, :8]
        rows[k] -= rows[k - 2]
    rows[0, 0] = 0.5
    return rows.astype(np.float32)


_S_MONO = _s_mono_matrix()


def _eye_block(j):
    row = jax.lax.broadcasted_iota(jnp.int32, (_N, _CB), 0)
    col = jax.lax.broadcasted_iota(jnp.int32, (_N, _CB), 1) + j * _CB
    return jnp.where(row == col, jnp.float32(1.0), jnp.float32(0.0))


def _pow_body(a_ref, f2_ref, f3_ref):
    j = pl.program_id(0)
    A = a_ref[...]
    Ab = a_ref[:, pl.ds(j * _CB, _CB)]
    eye = _eye_block(j)
    Fb = 2.0 * Ab - 2.0 * eye
    F2b = 2.0 * jnp.dot(A, Fb, preferred_element_type=jnp.float32) - 2.0 * Fb
    F3b = 2.0 * jnp.dot(A, F2b, preferred_element_type=jnp.float32) - 2.0 * F2b
    f2_ref[...] = F2b
    f3_ref[...] = F3b


def _thin_body(a_ref, dc_ref, c_ref, corr_ref):
    A = a_ref[...]
    e1 = dc_ref[...]
    corr = e1 * c_ref[1]
    eold = jnp.zeros((_N, _W), jnp.float32)
    ecur = e1
    for k in range(2, _M + 1):
        enew = 2.0 * jnp.dot(A, ecur, preferred_element_type=jnp.float32) - 2.0 * ecur - eold
        corr = corr + enew * c_ref[k]
        eold = ecur
        ecur = enew
    corr_ref[...] = corr


def _f4_body(f2full_ref, ab_ref, f3b_ref, p_ref, f4_ref, t_ref):
    j = pl.program_id(0)
    F2 = f2full_ref[...]
    F2b = f2full_ref[:, pl.ds(j * _CB, _CB)]
    F4b = jnp.dot(F2, F2b, preferred_element_type=jnp.float32)
    eye = _eye_block(j)
    Fb = 2.0 * ab_ref[...] - 2.0 * eye
    Tb = (p_ref[4] * eye + p_ref[5] * Fb + p_ref[6] * F2b
          + p_ref[7] * f3b_ref[...] + p_ref[8] * F4b)
    f4_ref[...] = F4b
    t_ref[...] = Tb


def _fin_body(f4full_ref, tb_ref, ab_ref, f2b_ref, f3b_ref, corr_ref, selb_ref, p_ref, r_ref):
    j = pl.program_id(0)
    F4 = f4full_ref[...]
    Yb = jnp.dot(F4, tb_ref[...], preferred_element_type=jnp.float32)
    eye = _eye_block(j)
    Fb = 2.0 * ab_ref[...] - 2.0 * eye
    Cb = jnp.dot(corr_ref[...], selb_ref[...], preferred_element_type=jnp.float32)
    r_ref[...] = (p_ref[0] * eye + p_ref[1] * Fb + p_ref[2] * f2b_ref[...]
                  + p_ref[3] * f3b_ref[...] + Yb + Cb)


def _full_spec():
    return pl.BlockSpec((_N, _N), lambda j: (0, 0))


def _blk_spec():
    return pl.BlockSpec((_N, _CB), lambda j: (0, j))


def _smem_spec():
    return pl.BlockSpec(memory_space=pltpu.SMEM)


def _cheb_compute(A_sum, Dc, Sel, c, p):
    nblk = _N // _CB
    F2, F3 = pl.pallas_call(
        _pow_body,
        grid=(nblk,),
        in_specs=[_full_spec()],
        out_specs=[_blk_spec(), _blk_spec()],
        out_shape=[jax.ShapeDtypeStruct((_N, _N), jnp.float32)] * 2,
    )(A_sum)
    corr = pl.pallas_call(
        _thin_body,
        in_specs=[pl.BlockSpec((_N, _N), lambda: (0, 0)),
                  pl.BlockSpec((_N, _W), lambda: (0, 0)),
                  _smem_spec()],
        out_specs=pl.BlockSpec((_N, _W), lambda: (0, 0)),
        out_shape=jax.ShapeDtypeStruct((_N, _W), jnp.float32),
    )(A_sum, Dc, c)
    F4, T = pl.pallas_call(
        _f4_body,
        grid=(nblk,),
        in_specs=[_full_spec(), _blk_spec(), _blk_spec(), _smem_spec()],
        out_specs=[_blk_spec(), _blk_spec()],
        out_shape=[jax.ShapeDtypeStruct((_N, _N), jnp.float32)] * 2,
    )(F2, A_sum, F3, p)
    r = pl.pallas_call(
        _fin_body,
        grid=(nblk,),
        in_specs=[_full_spec(), _blk_spec(), _blk_spec(), _blk_spec(), _blk_spec(),
                  pl.BlockSpec((_N, _W), lambda j: (0, 0)),
                  pl.BlockSpec((_W, _CB), lambda j: (0, j)),
                  _smem_spec()],
        out_specs=_blk_spec(),
        out_shape=jax.ShapeDtypeStruct((_N, _N), jnp.float32),
    )(F4, T, A_sum, F2, F3, corr, Sel, p)
    return r


@jax.jit
def _run(edge_index, edge_weight):
    c = _coeffs()
    p = c @ jnp.asarray(_S_MONO)
    A_set = jnp.zeros((_N, _N), jnp.float32).at[edge_index[0], edge_index[1]].set(edge_weight)
    A_sum = jnp.zeros((_N, _N), jnp.float32).at[edge_index[0], edge_index[1]].add(edge_weight)
    D = A_set - A_sum
    colmask = jnp.any(D != 0.0, axis=0)
    cols = jnp.nonzero(colmask, size=_W, fill_value=0)[0]
    valid = jnp.arange(_W) < jnp.sum(colmask)
    Dc = jnp.where(valid[None, :], D[:, cols], 0.0)
    Sel = ((cols[:, None] == jnp.arange(_N)[None, :]) & valid[:, None]).astype(jnp.float32)
    return _cheb_compute(A_sum, Dc, Sel, c, p)


def kernel(edge_index, edge_weight):
    return _run(edge_index, edge_weight)


# speedup vs baseline: 1.0841x; 1.0521x over previous
import functools

import jax
import jax.numpy as jnp
import numpy as np
from jax import lax
from jax.experimental import pallas as pl
from jax.experimental.pallas import tpu as pltpu
from jax.experimental.pallas import tpu_sc as plsc

_N = 2048
_E = 32768
_M = 8
_NPTS = _M + 1
_CB = 256
_W = 256

_NTILES = 32
_RPP = 16
_PASSES = 4
_ECHUNK = 4096
_NCHUNK = _E // _ECHUNK


def _coeffs():
    tmpN = jnp.arange(_NPTS, dtype=jnp.float32)
    num = jnp.cos(jnp.pi * (tmpN + 0.5) / _NPTS)
    kx = jnp.exp(-1.0 * (1.0 * num + 1.0)).reshape(-1, 1)
    rows = []
    for o in range(_M + 1):
        rows.append((2.0 / _NPTS) * (jnp.cos(jnp.pi * o * (tmpN + 0.5) / _NPTS).reshape(1, -1) @ kx))
    return jnp.concatenate(rows, axis=0)[:, 0]


def _s_mono_matrix():
    rows = np.zeros((9, 9))
    rows[0, 0] = 1.0
    rows[1, 1] = 0.5
    for k in range(2, 9):
        rows[k, 1:] = rows[k - 1, :8]
        rows[k] -= rows[k - 2]
    rows[0, 0] = 0.5
    return rows.astype(np.float32)


_S_MONO = _s_mono_matrix()


def _sc_scatter_body(ei_hbm, ew_hbm, zeros_hbm,
                     asum_hbm, aset_hbm,
                     acc_sum, acc_set, iv, jv, wv, sem):
    wid = lax.axis_index("s") * 2 + lax.axis_index("c")
    for p in range(_PASSES):
        base = wid * (_RPP * _PASSES) + p * _RPP
        pltpu.sync_copy(zeros_hbm, acc_sum)
        pltpu.sync_copy(zeros_hbm, acc_set)
        for ch in range(_NCHUNK):
            pltpu.sync_copy(ei_hbm.at[0, pl.ds(ch * _ECHUNK, _ECHUNK)], iv)
            pltpu.sync_copy(ei_hbm.at[1, pl.ds(ch * _ECHUNK, _ECHUNK)], jv)
            pltpu.sync_copy(ew_hbm.at[pl.ds(ch * _ECHUNK, _ECHUNK)], wv)

            def body(k, _, base=base):
                ivv = iv[pl.ds(k * 16, 16)]
                jvv = jv[pl.ds(k * 16, 16)]
                wvv = wv[pl.ds(k * 16, 16)]
                li = ivv - base
                m = (li >= 0) & (li < _RPP)
                flat = li * _N + jvv
                plsc.addupdate_scatter(acc_sum, [flat], wvv, mask=m)
                plsc.store_scatter(acc_set, [flat], wvv, mask=m)
                return 0

            lax.fori_loop(0, _ECHUNK // 16, body, 0)
        pltpu.sync_copy(acc_sum, asum_hbm.at[pl.ds(base * _N, _RPP * _N)])
        pltpu.sync_copy(acc_set, aset_hbm.at[pl.ds(base * _N, _RPP * _N)])


def _sc_scatter(edge_index, edge_weight, zeros_blk):
    mesh = plsc.VectorSubcoreMesh(core_axis_name="c", subcore_axis_name="s")
    k = functools.partial(
        pl.kernel,
        mesh=mesh,
        compiler_params=pltpu.CompilerParams(needs_layout_passes=False),
        out_type=[jax.ShapeDtypeStruct((_N * _N,), jnp.float32),
                  jax.ShapeDtypeStruct((_N * _N,), jnp.float32)],
        scratch_types=[
            pltpu.VMEM((_RPP * _N,), jnp.float32),
            pltpu.VMEM((_RPP * _N,), jnp.float32),
            pltpu.VMEM((_ECHUNK,), jnp.int32),
            pltpu.VMEM((_ECHUNK,), jnp.int32),
            pltpu.VMEM((_ECHUNK,), jnp.float32),
            pltpu.SemaphoreType.DMA,
        ],
    )(_sc_scatter_body)
    a_sum, a_set = k(edge_index, edge_weight, zeros_blk)
    return a_sum.reshape(_N, _N), a_set.reshape(_N, _N)


def _eye_block(j):
    row = jax.lax.broadcasted_iota(jnp.int32, (_N, _CB), 0)
    col = jax.lax.broadcasted_iota(jnp.int32, (_N, _CB), 1) + j * _CB
    return jnp.where(row == col, jnp.float32(1.0), jnp.float32(0.0))


def _pow_body(a_ref, f2_ref, f3_ref):
    j = pl.program_id(0)
    A = a_ref[...]
    Ab = a_ref[:, pl.ds(j * _CB, _CB)]
    eye = _eye_block(j)
    Fb = 2.0 * Ab - 2.0 * eye
    F2b = 2.0 * jnp.dot(A, Fb, preferred_element_type=jnp.float32) - 2.0 * Fb
    F3b = 2.0 * jnp.dot(A, F2b, preferred_element_type=jnp.float32) - 2.0 * F2b
    f2_ref[...] = F2b
    f3_ref[...] = F3b


def _thin_body(a_ref, dc_ref, c_ref, corr_ref):
    A = a_ref[...]
    e1 = dc_ref[...]
    corr = e1 * c_ref[1]
    eold = jnp.zeros((_N, _W), jnp.float32)
    ecur = e1
    for k in range(2, _M + 1):
        enew = 2.0 * jnp.dot(A, ecur, preferred_element_type=jnp.float32) - 2.0 * ecur - eold
        corr = corr + enew * c_ref[k]
        eold = ecur
        ecur = enew
    corr_ref[...] = corr


def _f4_body(f2full_ref, ab_ref, f3b_ref, p_ref, f4_ref, t_ref):
    j = pl.program_id(0)
    F2 = f2full_ref[...]
    F2b = f2full_ref[:, pl.ds(j * _CB, _CB)]
    F4b = jnp.dot(F2, F2b, preferred_element_type=jnp.float32)
    eye = _eye_block(j)
    Fb = 2.0 * ab_ref[...] - 2.0 * eye
    Tb = (p_ref[4] * eye + p_ref[5] * Fb + p_ref[6] * F2b
          + p_ref[7] * f3b_ref[...] + p_ref[8] * F4b)
    f4_ref[...] = F4b
    t_ref[...] = Tb


def _fin_body(f4full_ref, tb_ref, ab_ref, f2b_ref, f3b_ref, corr_ref, selb_ref, p_ref, r_ref):
    j = pl.program_id(0)
    F4 = f4full_ref[...]
    Yb = jnp.dot(F4, tb_ref[...], preferred_element_type=jnp.float32)
    eye = _eye_block(j)
    Fb = 2.0 * ab_ref[...] - 2.0 * eye
    Cb = jnp.dot(corr_ref[...], selb_ref[...], preferred_element_type=jnp.float32)
    r_ref[...] = (p_ref[0] * eye + p_ref[1] * Fb + p_ref[2] * f2b_ref[...]
                  + p_ref[3] * f3b_ref[...] + Yb + Cb)


def _full_spec():
    return pl.BlockSpec((_N, _N), lambda j: (0, 0))


def _blk_spec():
    return pl.BlockSpec((_N, _CB), lambda j: (0, j))


def _smem_spec():
    return pl.BlockSpec(memory_space=pltpu.SMEM)


def _cheb_compute(A_sum, Dc, Sel, c, p):
    nblk = _N // _CB
    F2, F3 = pl.pallas_call(
        _pow_body,
        grid=(nblk,),
        in_specs=[_full_spec()],
        out_specs=[_blk_spec(), _blk_spec()],
        out_shape=[jax.ShapeDtypeStruct((_N, _N), jnp.float32)] * 2,
    )(A_sum)
    corr = pl.pallas_call(
        _thin_body,
        in_specs=[pl.BlockSpec((_N, _N), lambda: (0, 0)),
                  pl.BlockSpec((_N, _W), lambda: (0, 0)),
                  _smem_spec()],
        out_specs=pl.BlockSpec((_N, _W), lambda: (0, 0)),
        out_shape=jax.ShapeDtypeStruct((_N, _W), jnp.float32),
    )(A_sum, Dc, c)
    F4, T = pl.pallas_call(
        _f4_body,
        grid=(nblk,),
        in_specs=[_full_spec(), _blk_spec(), _blk_spec(), _smem_spec()],
        out_specs=[_blk_spec(), _blk_spec()],
        out_shape=[jax.ShapeDtypeStruct((_N, _N), jnp.float32)] * 2,
    )(F2, A_sum, F3, p)
    r = pl.pallas_call(
        _fin_body,
        grid=(nblk,),
        in_specs=[_full_spec(), _blk_spec(), _blk_spec(), _blk_spec(), _blk_spec(),
                  pl.BlockSpec((_N, _W), lambda j: (0, 0)),
                  pl.BlockSpec((_W, _CB), lambda j: (0, j)),
                  _smem_spec()],
        out_specs=_blk_spec(),
        out_shape=jax.ShapeDtypeStruct((_N, _N), jnp.float32),
    )(F4, T, A_sum, F2, F3, corr, Sel, p)
    return r


@jax.jit
def _run(edge_index, edge_weight):
    c = _coeffs()
    p = c @ jnp.asarray(_S_MONO)
    zeros_blk = jnp.zeros((_RPP * _N,), jnp.float32)
    A_sum, A_set = _sc_scatter(edge_index, edge_weight, zeros_blk)
    D = A_set - A_sum
    colmask = jnp.any(D != 0.0, axis=0)
    cols = jnp.nonzero(colmask, size=_W, fill_value=0)[0]
    valid = jnp.arange(_W) < jnp.sum(colmask)
    Dc = jnp.where(valid[None, :], D[:, cols], 0.0)
    Sel = ((cols[:, None] == jnp.arange(_N)[None, :]) & valid[:, None]).astype(jnp.float32)
    return _cheb_compute(A_sum, Dc, Sel, c, p)


def kernel(edge_index, edge_weight):
    return _run(edge_index, edge_weight)


# speedup vs baseline: 1.2075x; 1.1138x over previous
import functools

import jax
import jax.numpy as jnp
import numpy as np
from jax import lax
from jax.experimental import pallas as pl
from jax.experimental.pallas import tpu as pltpu
from jax.experimental.pallas import tpu_sc as plsc

_N = 2048
_E = 32768
_M = 8
_NPTS = _M + 1
_CB = 256
_W = 256

_NTILES = 32
_RPP = 16
_PASSES = 4
_ECHUNK = 4096
_NCHUNK = _E // _ECHUNK


def _coeffs():
    tmpN = jnp.arange(_NPTS, dtype=jnp.float32)
    num = jnp.cos(jnp.pi * (tmpN + 0.5) / _NPTS)
    kx = jnp.exp(-1.0 * (1.0 * num + 1.0)).reshape(-1, 1)
    rows = []
    for o in range(_M + 1):
        rows.append((2.0 / _NPTS) * (jnp.cos(jnp.pi * o * (tmpN + 0.5) / _NPTS).reshape(1, -1) @ kx))
    return jnp.concatenate(rows, axis=0)[:, 0]


def _s_mono_matrix():
    rows = np.zeros((9, 9))
    rows[0, 0] = 1.0
    rows[1, 1] = 0.5
    for k in range(2, 9):
        rows[k, 1:] = rows[k - 1, :8]
        rows[k] -= rows[k - 2]
    rows[0, 0] = 0.5
    return rows.astype(np.float32)


_S_MONO = _s_mono_matrix()


_UNROLL = 8


def _sc_scatter_body(ei_hbm, ew_hbm, zeros_hbm,
                     asum_hbm, aset_hbm,
                     acc_sum, acc_set, iv, jv, wv, sem0, sem1, semz):
    wid = lax.axis_index("s") * 2 + lax.axis_index("c")
    sems = [sem0, sem1]

    def start_chunk(ch, b):
        return (pltpu.async_copy(ei_hbm.at[0, pl.ds(ch * _ECHUNK, _ECHUNK)], iv.at[b], sems[b]),
                pltpu.async_copy(ei_hbm.at[1, pl.ds(ch * _ECHUNK, _ECHUNK)], jv.at[b], sems[b]),
                pltpu.async_copy(ew_hbm.at[pl.ds(ch * _ECHUNK, _ECHUNK)], wv.at[b], sems[b]))

    for p in range(_PASSES):
        base = wid * (_RPP * _PASSES) + p * _RPP
        hz0 = pltpu.async_copy(zeros_hbm, acc_sum, semz)
        hz1 = pltpu.async_copy(zeros_hbm, acc_set, semz)
        pending = start_chunk(0, 0)
        hz0.wait()
        hz1.wait()
        for ch in range(_NCHUNK):
            b = ch % 2
            if ch + 1 < _NCHUNK:
                nxt = start_chunk(ch + 1, 1 - b)
            for h in pending:
                h.wait()

            def body(k, _, base=base, b=b):
                for u in range(_UNROLL):
                    off = (k * _UNROLL + u) * 16
                    ivv = iv[b, pl.ds(off, 16)]
                    jvv = jv[b, pl.ds(off, 16)]
                    wvv = wv[b, pl.ds(off, 16)]
                    li = ivv - base
                    m = (li >= 0) & (li < _RPP)
                    flat = li * _N + jvv
                    plsc.addupdate_scatter(acc_sum, [flat], wvv, mask=m)
                    plsc.store_scatter(acc_set, [flat], wvv, mask=m)
                return 0

            lax.fori_loop(0, _ECHUNK // (16 * _UNROLL), body, 0)
            if ch + 1 < _NCHUNK:
                pending = nxt
        pltpu.sync_copy(acc_sum, asum_hbm.at[pl.ds(base * _N, _RPP * _N)])
        pltpu.sync_copy(acc_set, aset_hbm.at[pl.ds(base * _N, _RPP * _N)])


def _sc_scatter(edge_index, edge_weight, zeros_blk):
    mesh = plsc.VectorSubcoreMesh(core_axis_name="c", subcore_axis_name="s")
    k = functools.partial(
        pl.kernel,
        mesh=mesh,
        compiler_params=pltpu.CompilerParams(needs_layout_passes=False),
        out_type=[jax.ShapeDtypeStruct((_N * _N,), jnp.float32),
                  jax.ShapeDtypeStruct((_N * _N,), jnp.float32)],
        scratch_types=[
            pltpu.VMEM((_RPP * _N,), jnp.float32),
            pltpu.VMEM((_RPP * _N,), jnp.float32),
            pltpu.VMEM((2, _ECHUNK), jnp.int32),
            pltpu.VMEM((2, _ECHUNK), jnp.int32),
            pltpu.VMEM((2, _ECHUNK), jnp.float32),
            pltpu.SemaphoreType.DMA,
            pltpu.SemaphoreType.DMA,
            pltpu.SemaphoreType.DMA,
        ],
    )(_sc_scatter_body)
    a_sum, a_set = k(edge_index, edge_weight, zeros_blk)
    return a_sum.reshape(_N, _N), a_set.reshape(_N, _N)


def _eye_block(j):
    row = jax.lax.broadcasted_iota(jnp.int32, (_N, _CB), 0)
    col = jax.lax.broadcasted_iota(jnp.int32, (_N, _CB), 1) + j * _CB
    return jnp.where(row == col, jnp.float32(1.0), jnp.float32(0.0))


def _pow_body(a_ref, f2_ref, f3_ref):
    j = pl.program_id(0)
    A = a_ref[...]
    Ab = a_ref[:, pl.ds(j * _CB, _CB)]
    eye = _eye_block(j)
    Fb = 2.0 * Ab - 2.0 * eye
    F2b = 2.0 * jnp.dot(A, Fb, preferred_element_type=jnp.float32) - 2.0 * Fb
    F3b = 2.0 * jnp.dot(A, F2b, preferred_element_type=jnp.float32) - 2.0 * F2b
    f2_ref[...] = F2b
    f3_ref[...] = F3b


def _thin_body(a_ref, dc_ref, c_ref, corr_ref):
    A = a_ref[...]
    e1 = dc_ref[...]
    corr = e1 * c_ref[1]
    eold = jnp.zeros((_N, _W), jnp.float32)
    ecur = e1
    for k in range(2, _M + 1):
        enew = 2.0 * jnp.dot(A, ecur, preferred_element_type=jnp.float32) - 2.0 * ecur - eold
        corr = corr + enew * c_ref[k]
        eold = ecur
        ecur = enew
    corr_ref[...] = corr


def _f4_body(f2full_ref, ab_ref, f3b_ref, p_ref, f4_ref, t_ref):
    j = pl.program_id(0)
    F2 = f2full_ref[...]
    F2b = f2full_ref[:, pl.ds(j * _CB, _CB)]
    F4b = jnp.dot(F2, F2b, preferred_element_type=jnp.float32)
    eye = _eye_block(j)
    Fb = 2.0 * ab_ref[...] - 2.0 * eye
    Tb = (p_ref[4] * eye + p_ref[5] * Fb + p_ref[6] * F2b
          + p_ref[7] * f3b_ref[...] + p_ref[8] * F4b)
    f4_ref[...] = F4b
    t_ref[...] = Tb


def _fin_body(f4full_ref, tb_ref, ab_ref, f2b_ref, f3b_ref, corr_ref, selb_ref, p_ref, r_ref):
    j = pl.program_id(0)
    F4 = f4full_ref[...]
    Yb = jnp.dot(F4, tb_ref[...], preferred_element_type=jnp.float32)
    eye = _eye_block(j)
    Fb = 2.0 * ab_ref[...] - 2.0 * eye
    Cb = jnp.dot(corr_ref[...], selb_ref[...], preferred_element_type=jnp.float32)
    r_ref[...] = (p_ref[0] * eye + p_ref[1] * Fb + p_ref[2] * f2b_ref[...]
                  + p_ref[3] * f3b_ref[...] + Yb + Cb)


def _full_spec():
    return pl.BlockSpec((_N, _N), lambda j: (0, 0))


def _blk_spec():
    return pl.BlockSpec((_N, _CB), lambda j: (0, j))


def _smem_spec():
    return pl.BlockSpec(memory_space=pltpu.SMEM)


def _cheb_compute(A_sum, Dc, Sel, c, p):
    nblk = _N // _CB
    F2, F3 = pl.pallas_call(
        _pow_body,
        grid=(nblk,),
        in_specs=[_full_spec()],
        out_specs=[_blk_spec(), _blk_spec()],
        out_shape=[jax.ShapeDtypeStruct((_N, _N), jnp.float32)] * 2,
    )(A_sum)
    corr = pl.pallas_call(
        _thin_body,
        in_specs=[pl.BlockSpec((_N, _N), lambda: (0, 0)),
                  pl.BlockSpec((_N, _W), lambda: (0, 0)),
                  _smem_spec()],
        out_specs=pl.BlockSpec((_N, _W), lambda: (0, 0)),
        out_shape=jax.ShapeDtypeStruct((_N, _W), jnp.float32),
    )(A_sum, Dc, c)
    F4, T = pl.pallas_call(
        _f4_body,
        grid=(nblk,),
        in_specs=[_full_spec(), _blk_spec(), _blk_spec(), _smem_spec()],
        out_specs=[_blk_spec(), _blk_spec()],
        out_shape=[jax.ShapeDtypeStruct((_N, _N), jnp.float32)] * 2,
    )(F2, A_sum, F3, p)
    r = pl.pallas_call(
        _fin_body,
        grid=(nblk,),
        in_specs=[_full_spec(), _blk_spec(), _blk_spec(), _blk_spec(), _blk_spec(),
                  pl.BlockSpec((_N, _W), lambda j: (0, 0)),
                  pl.BlockSpec((_W, _CB), lambda j: (0, j)),
                  _smem_spec()],
        out_specs=_blk_spec(),
        out_shape=jax.ShapeDtypeStruct((_N, _N), jnp.float32),
    )(F4, T, A_sum, F2, F3, corr, Sel, p)
    return r


@jax.jit
def _run(edge_index, edge_weight):
    c = _coeffs()
    p = c @ jnp.asarray(_S_MONO)
    zeros_blk = jnp.zeros((_RPP * _N,), jnp.float32)
    A_sum, A_set = _sc_scatter(edge_index, edge_weight, zeros_blk)
    colmask = jnp.any(A_set != A_sum, axis=0)
    cols = jnp.nonzero(colmask, size=_W, fill_value=0)[0]
    valid = jnp.arange(_W) < jnp.sum(colmask)
    Dc = jnp.where(valid[None, :], A_set[:, cols] - A_sum[:, cols], 0.0)
    Sel = ((cols[:, None] == jnp.arange(_N)[None, :]) & valid[:, None]).astype(jnp.float32)
    return _cheb_compute(A_sum, Dc, Sel, c, p)


def kernel(edge_index, edge_weight):
    return _run(edge_index, edge_weight)


# speedup vs baseline: 1.2124x; 1.0041x over previous
import functools

import jax
import jax.numpy as jnp
import numpy as np
from jax import lax
from jax.experimental import pallas as pl
from jax.experimental.pallas import tpu as pltpu
from jax.experimental.pallas import tpu_sc as plsc

_N = 2048
_E = 32768
_M = 8
_NPTS = _M + 1
_CB = 256
_W = 256

_NTILES = 32
_RPP = 16
_PASSES = 4
_ECHUNK = 4096
_NCHUNK = _E // _ECHUNK


def _coeffs():
    tmpN = jnp.arange(_NPTS, dtype=jnp.float32)
    num = jnp.cos(jnp.pi * (tmpN + 0.5) / _NPTS)
    kx = jnp.exp(-1.0 * (1.0 * num + 1.0)).reshape(-1, 1)
    rows = []
    for o in range(_M + 1):
        rows.append((2.0 / _NPTS) * (jnp.cos(jnp.pi * o * (tmpN + 0.5) / _NPTS).reshape(1, -1) @ kx))
    return jnp.concatenate(rows, axis=0)[:, 0]


def _s_mono_matrix():
    rows = np.zeros((9, 9))
    rows[0, 0] = 1.0
    rows[1, 1] = 0.5
    for k in range(2, 9):
        rows[k, 1:] = rows[k - 1, :8]
        rows[k] -= rows[k - 2]
    rows[0, 0] = 0.5
    return rows.astype(np.float32)


_S_MONO = _s_mono_matrix()


_UNROLL = 8


def _sc_scatter_body(key_hbm, ew_hbm, zeros_hbm,
                     asum_hbm, aset_hbm,
                     acc_sum, acc_set, kv, wv, sem0, sem1, semz):
    wid = lax.axis_index("s") * 2 + lax.axis_index("c")
    sems = [sem0, sem1]

    def start_chunk(ch, b):
        return (pltpu.async_copy(key_hbm.at[pl.ds(ch * _ECHUNK, _ECHUNK)], kv.at[b], sems[b]),
                pltpu.async_copy(ew_hbm.at[pl.ds(ch * _ECHUNK, _ECHUNK)], wv.at[b], sems[b]))

    for p in range(_PASSES):
        base = wid * (_RPP * _PASSES) + p * _RPP
        hz0 = pltpu.async_copy(zeros_hbm, acc_sum, semz)
        hz1 = pltpu.async_copy(zeros_hbm, acc_set, semz)
        pending = start_chunk(0, 0)
        hz0.wait()
        hz1.wait()
        for ch in range(_NCHUNK):
            b = ch % 2
            if ch + 1 < _NCHUNK:
                nxt = start_chunk(ch + 1, 1 - b)
            for h in pending:
                h.wait()

            lo = base * _N

            def body(k, _, lo=lo, b=b):
                for u in range(_UNROLL):
                    off = (k * _UNROLL + u) * 16
                    kvv = kv[b, pl.ds(off, 16)]
                    wvv = wv[b, pl.ds(off, 16)]
                    flat = kvv - lo
                    m = (flat >= 0) & (flat < _RPP * _N)
                    plsc.addupdate_scatter(acc_sum, [flat], wvv, mask=m)
                    plsc.store_scatter(acc_set, [flat], wvv, mask=m)
                return 0

            lax.fori_loop(0, _ECHUNK // (16 * _UNROLL), body, 0)
            if ch + 1 < _NCHUNK:
                pending = nxt
        pltpu.sync_copy(acc_sum, asum_hbm.at[pl.ds(base * _N, _RPP * _N)])
        pltpu.sync_copy(acc_set, aset_hbm.at[pl.ds(base * _N, _RPP * _N)])


def _sc_scatter(edge_index, edge_weight, zeros_blk):
    mesh = plsc.VectorSubcoreMesh(core_axis_name="c", subcore_axis_name="s")
    k = functools.partial(
        pl.kernel,
        mesh=mesh,
        compiler_params=pltpu.CompilerParams(needs_layout_passes=False),
        out_type=[jax.ShapeDtypeStruct((_N * _N,), jnp.float32),
                  jax.ShapeDtypeStruct((_N * _N,), jnp.float32)],
        scratch_types=[
            pltpu.VMEM((_RPP * _N,), jnp.float32),
            pltpu.VMEM((_RPP * _N,), jnp.float32),
            pltpu.VMEM((2, _ECHUNK), jnp.int32),
            pltpu.VMEM((2, _ECHUNK), jnp.float32),
            pltpu.SemaphoreType.DMA,
            pltpu.SemaphoreType.DMA,
            pltpu.SemaphoreType.DMA,
        ],
    )(_sc_scatter_body)
    flatkey = edge_index[0] * _N + edge_index[1]
    a_sum, a_set = k(flatkey, edge_weight, zeros_blk)
    return a_sum.reshape(_N, _N), a_set.reshape(_N, _N)


def _eye_block(j):
    row = jax.lax.broadcasted_iota(jnp.int32, (_N, _CB), 0)
    col = jax.lax.broadcasted_iota(jnp.int32, (_N, _CB), 1) + j * _CB
    return jnp.where(row == col, jnp.float32(1.0), jnp.float32(0.0))


def _pow_body(a_ref, f2_ref, f3_ref):
    j = pl.program_id(0)
    A = a_ref[...]
    Ab = a_ref[:, pl.ds(j * _CB, _CB)]
    eye = _eye_block(j)
    Fb = 2.0 * Ab - 2.0 * eye
    F2b = 2.0 * jnp.dot(A, Fb, preferred_element_type=jnp.float32) - 2.0 * Fb
    F3b = 2.0 * jnp.dot(A, F2b, preferred_element_type=jnp.float32) - 2.0 * F2b
    f2_ref[...] = F2b
    f3_ref[...] = F3b


def _thin_body(a_ref, dc_ref, c_ref, corr_ref):
    A = a_ref[...]
    e1 = dc_ref[...]
    corr = e1 * c_ref[1]
    eold = jnp.zeros((_N, _W), jnp.float32)
    ecur = e1
    for k in range(2, _M + 1):
        enew = 2.0 * jnp.dot(A, ecur, preferred_element_type=jnp.float32) - 2.0 * ecur - eold
        corr = corr + enew * c_ref[k]
        eold = ecur
        ecur = enew
    corr_ref[...] = corr


def _f4_body(f2full_ref, ab_ref, f3b_ref, p_ref, f4_ref, t_ref):
    j = pl.program_id(0)
    F2 = f2full_ref[...]
    F2b = f2full_ref[:, pl.ds(j * _CB, _CB)]
    F4b = jnp.dot(F2, F2b, preferred_element_type=jnp.float32)
    eye = _eye_block(j)
    Fb = 2.0 * ab_ref[...] - 2.0 * eye
    Tb = (p_ref[4] * eye + p_ref[5] * Fb + p_ref[6] * F2b
          + p_ref[7] * f3b_ref[...] + p_ref[8] * F4b)
    f4_ref[...] = F4b
    t_ref[...] = Tb


def _fin_body(f4full_ref, tb_ref, ab_ref, f2b_ref, f3b_ref, corr_ref, selb_ref, p_ref, r_ref):
    j = pl.program_id(0)
    F4 = f4full_ref[...]
    Yb = jnp.dot(F4, tb_ref[...], preferred_element_type=jnp.float32)
    eye = _eye_block(j)
    Fb = 2.0 * ab_ref[...] - 2.0 * eye
    Cb = jnp.dot(corr_ref[...], selb_ref[...], preferred_element_type=jnp.float32)
    r_ref[...] = (p_ref[0] * eye + p_ref[1] * Fb + p_ref[2] * f2b_ref[...]
                  + p_ref[3] * f3b_ref[...] + Yb + Cb)


def _full_spec():
    return pl.BlockSpec((_N, _N), lambda j: (0, 0))


def _blk_spec():
    return pl.BlockSpec((_N, _CB), lambda j: (0, j))


def _smem_spec():
    return pl.BlockSpec(memory_space=pltpu.SMEM)


def _cheb_compute(A_sum, Dc, Sel, c, p):
    nblk = _N // _CB
    F2, F3 = pl.pallas_call(
        _pow_body,
        grid=(nblk,),
        in_specs=[_full_spec()],
        out_specs=[_blk_spec(), _blk_spec()],
        out_shape=[jax.ShapeDtypeStruct((_N, _N), jnp.float32)] * 2,
    )(A_sum)
    corr = pl.pallas_call(
        _thin_body,
        in_specs=[pl.BlockSpec((_N, _N), lambda: (0, 0)),
                  pl.BlockSpec((_N, _W), lambda: (0, 0)),
                  _smem_spec()],
        out_specs=pl.BlockSpec((_N, _W), lambda: (0, 0)),
        out_shape=jax.ShapeDtypeStruct((_N, _W), jnp.float32),
    )(A_sum, Dc, c)
    F4, T = pl.pallas_call(
        _f4_body,
        grid=(nblk,),
        in_specs=[_full_spec(), _blk_spec(), _blk_spec(), _smem_spec()],
        out_specs=[_blk_spec(), _blk_spec()],
        out_shape=[jax.ShapeDtypeStruct((_N, _N), jnp.float32)] * 2,
    )(F2, A_sum, F3, p)
    r = pl.pallas_call(
        _fin_body,
        grid=(nblk,),
        in_specs=[_full_spec(), _blk_spec(), _blk_spec(), _blk_spec(), _blk_spec(),
                  pl.BlockSpec((_N, _W), lambda j: (0, 0)),
                  pl.BlockSpec((_W, _CB), lambda j: (0, j)),
                  _smem_spec()],
        out_specs=_blk_spec(),
        out_shape=jax.ShapeDtypeStruct((_N, _N), jnp.float32),
    )(F4, T, A_sum, F2, F3, corr, Sel, p)
    return r


@jax.jit
def _run(edge_index, edge_weight):
    c = _coeffs()
    p = c @ jnp.asarray(_S_MONO)
    zeros_blk = jnp.zeros((_RPP * _N,), jnp.float32)
    A_sum, A_set = _sc_scatter(edge_index, edge_weight, zeros_blk)
    colmask = jnp.any(A_set != A_sum, axis=0)
    cols = jnp.nonzero(colmask, size=_W, fill_value=0)[0]
    valid = jnp.arange(_W) < jnp.sum(colmask)
    Dc = jnp.where(valid[None, :], A_set[:, cols] - A_sum[:, cols], 0.0)
    Sel = ((cols[:, None] == jnp.arange(_N)[None, :]) & valid[:, None]).astype(jnp.float32)
    return _cheb_compute(A_sum, Dc, Sel, c, p)


def kernel(edge_index, edge_weight):
    return _run(edge_index, edge_weight)


# speedup vs baseline: 1.2788x; 1.0548x over previous
import functools

import jax
import jax.numpy as jnp
import numpy as np
from jax import lax
from jax.experimental import pallas as pl
from jax.experimental.pallas import tpu as pltpu
from jax.experimental.pallas import tpu_sc as plsc

_N = 2048
_E = 32768
_M = 8
_NPTS = _M + 1
_CB = 256
_W = 256

_NTILES = 32
_RPP = 16
_PASSES = 4
_ECHUNK = 4096
_NCHUNK = _E // _ECHUNK


def _coeffs():
    tmpN = jnp.arange(_NPTS, dtype=jnp.float32)
    num = jnp.cos(jnp.pi * (tmpN + 0.5) / _NPTS)
    kx = jnp.exp(-1.0 * (1.0 * num + 1.0)).reshape(-1, 1)
    rows = []
    for o in range(_M + 1):
        rows.append((2.0 / _NPTS) * (jnp.cos(jnp.pi * o * (tmpN + 0.5) / _NPTS).reshape(1, -1) @ kx))
    return jnp.concatenate(rows, axis=0)[:, 0]


def _s_mono_matrix():
    rows = np.zeros((9, 9))
    rows[0, 0] = 1.0
    rows[1, 1] = 0.5
    for k in range(2, 9):
        rows[k, 1:] = rows[k - 1, :8]
        rows[k] -= rows[k - 2]
    rows[0, 0] = 0.5
    return rows.astype(np.float32)


_S_MONO = _s_mono_matrix()


_UNROLL = 16


def _sc_scatter_body(key_hbm, ew_hbm, zeros_hbm,
                     asum_hbm, aset_hbm,
                     acc_sum, acc_set, kv, wv, sem0, sem1, semz):
    wid = lax.axis_index("s") * 2 + lax.axis_index("c")
    sems = [sem0, sem1]

    def start_chunk(ch, b):
        return (pltpu.async_copy(key_hbm.at[pl.ds(ch * _ECHUNK, _ECHUNK)], kv.at[b], sems[b]),
                pltpu.async_copy(ew_hbm.at[pl.ds(ch * _ECHUNK, _ECHUNK)], wv.at[b], sems[b]))

    for p in range(_PASSES):
        base = wid * (_RPP * _PASSES) + p * _RPP
        hz0 = pltpu.async_copy(zeros_hbm, acc_sum, semz)
        hz1 = pltpu.async_copy(zeros_hbm, acc_set, semz)
        pending = start_chunk(0, 0)
        hz0.wait()
        hz1.wait()
        for ch in range(_NCHUNK):
            b = ch % 2
            if ch + 1 < _NCHUNK:
                nxt = start_chunk(ch + 1, 1 - b)
            for h in pending:
                h.wait()

            lo = base * _N

            def body(k, _, lo=lo, b=b):
                for u in range(_UNROLL):
                    off = (k * _UNROLL + u) * 16
                    kvv = kv[b, pl.ds(off, 16)]
                    wvv = wv[b, pl.ds(off, 16)]
                    flat = kvv - lo
                    m = (flat >= 0) & (flat < _RPP * _N)
                    li = lax.shift_right_logical(flat, 11)
                    colv = flat & (_N - 1)
                    plsc.addupdate_scatter(acc_sum, [li, colv], wvv, mask=m)
                    plsc.store_scatter(acc_set, [li, colv], wvv, mask=m)
                return 0

            lax.fori_loop(0, _ECHUNK // (16 * _UNROLL), body, 0)
            if ch + 1 < _NCHUNK:
                pending = nxt
        pltpu.sync_copy(acc_sum, asum_hbm.at[pl.ds(base, _RPP), :])
        pltpu.sync_copy(acc_set, aset_hbm.at[pl.ds(base, _RPP), :])


def _sc_scatter(edge_index, edge_weight, zeros_blk):
    mesh = plsc.VectorSubcoreMesh(core_axis_name="c", subcore_axis_name="s")
    k = functools.partial(
        pl.kernel,
        mesh=mesh,
        compiler_params=pltpu.CompilerParams(needs_layout_passes=False),
        out_type=[jax.ShapeDtypeStruct((_N, _N), jnp.float32),
                  jax.ShapeDtypeStruct((_N, _N), jnp.float32)],
        scratch_types=[
            pltpu.VMEM((_RPP, _N), jnp.float32),
            pltpu.VMEM((_RPP, _N), jnp.float32),
            pltpu.VMEM((2, _ECHUNK), jnp.int32),
            pltpu.VMEM((2, _ECHUNK), jnp.float32),
            pltpu.SemaphoreType.DMA,
            pltpu.SemaphoreType.DMA,
            pltpu.SemaphoreType.DMA,
        ],
    )(_sc_scatter_body)
    flatkey = edge_index[0] * _N + edge_index[1]
    return k(flatkey, edge_weight, zeros_blk)


def _eye_block(j):
    row = jax.lax.broadcasted_iota(jnp.int32, (_N, _CB), 0)
    col = jax.lax.broadcasted_iota(jnp.int32, (_N, _CB), 1) + j * _CB
    return jnp.where(row == col, jnp.float32(1.0), jnp.float32(0.0))


def _pow_body(a_ref, f2_ref, f3_ref):
    j = pl.program_id(0)
    A = a_ref[...]
    Ab = a_ref[:, pl.ds(j * _CB, _CB)]
    eye = _eye_block(j)
    Fb = 2.0 * Ab - 2.0 * eye
    F2b = 2.0 * jnp.dot(A, Fb, preferred_element_type=jnp.float32) - 2.0 * Fb
    F3b = 2.0 * jnp.dot(A, F2b, preferred_element_type=jnp.float32) - 2.0 * F2b
    f2_ref[...] = F2b
    f3_ref[...] = F3b


def _thin_body(a_ref, dc_ref, c_ref, corr_ref):
    A = a_ref[...]
    e1 = dc_ref[...]
    corr = e1 * c_ref[1]
    eold = jnp.zeros((_N, _W), jnp.float32)
    ecur = e1
    for k in range(2, _M + 1):
        enew = 2.0 * jnp.dot(A, ecur, preferred_element_type=jnp.float32) - 2.0 * ecur - eold
        corr = corr + enew * c_ref[k]
        eold = ecur
        ecur = enew
    corr_ref[...] = corr


def _f4_body(f2full_ref, ab_ref, f3b_ref, p_ref, f4_ref, t_ref):
    j = pl.program_id(0)
    F2 = f2full_ref[...]
    F2b = f2full_ref[:, pl.ds(j * _CB, _CB)]
    F4b = jnp.dot(F2, F2b, preferred_element_type=jnp.float32)
    eye = _eye_block(j)
    Fb = 2.0 * ab_ref[...] - 2.0 * eye
    Tb = (p_ref[4] * eye + p_ref[5] * Fb + p_ref[6] * F2b
          + p_ref[7] * f3b_ref[...] + p_ref[8] * F4b)
    f4_ref[...] = F4b
    t_ref[...] = Tb


def _fin_body(f4full_ref, tb_ref, ab_ref, f2b_ref, f3b_ref, corr_ref, selb_ref, p_ref, r_ref):
    j = pl.program_id(0)
    F4 = f4full_ref[...]
    Yb = jnp.dot(F4, tb_ref[...], preferred_element_type=jnp.float32)
    eye = _eye_block(j)
    Fb = 2.0 * ab_ref[...] - 2.0 * eye
    Cb = jnp.dot(corr_ref[...], selb_ref[...], preferred_element_type=jnp.float32)
    r_ref[...] = (p_ref[0] * eye + p_ref[1] * Fb + p_ref[2] * f2b_ref[...]
                  + p_ref[3] * f3b_ref[...] + Yb + Cb)


def _full_spec():
    return pl.BlockSpec((_N, _N), lambda j: (0, 0))


def _blk_spec():
    return pl.BlockSpec((_N, _CB), lambda j: (0, j))


def _smem_spec():
    return pl.BlockSpec(memory_space=pltpu.SMEM)


def _cheb_compute(A_sum, Dc, Sel, c, p):
    nblk = _N // _CB
    F2, F3 = pl.pallas_call(
        _pow_body,
        grid=(nblk,),
        in_specs=[_full_spec()],
        out_specs=[_blk_spec(), _blk_spec()],
        out_shape=[jax.ShapeDtypeStruct((_N, _N), jnp.float32)] * 2,
    )(A_sum)
    corr = pl.pallas_call(
        _thin_body,
        in_specs=[pl.BlockSpec((_N, _N), lambda: (0, 0)),
                  pl.BlockSpec((_N, _W), lambda: (0, 0)),
                  _smem_spec()],
        out_specs=pl.BlockSpec((_N, _W), lambda: (0, 0)),
        out_shape=jax.ShapeDtypeStruct((_N, _W), jnp.float32),
    )(A_sum, Dc, c)
    F4, T = pl.pallas_call(
        _f4_body,
        grid=(nblk,),
        in_specs=[_full_spec(), _blk_spec(), _blk_spec(), _smem_spec()],
        out_specs=[_blk_spec(), _blk_spec()],
        out_shape=[jax.ShapeDtypeStruct((_N, _N), jnp.float32)] * 2,
    )(F2, A_sum, F3, p)
    r = pl.pallas_call(
        _fin_body,
        grid=(nblk,),
        in_specs=[_full_spec(), _blk_spec(), _blk_spec(), _blk_spec(), _blk_spec(),
                  pl.BlockSpec((_N, _W), lambda j: (0, 0)),
                  pl.BlockSpec((_W, _CB), lambda j: (0, j)),
                  _smem_spec()],
        out_specs=_blk_spec(),
        out_shape=jax.ShapeDtypeStruct((_N, _N), jnp.float32),
    )(F4, T, A_sum, F2, F3, corr, Sel, p)
    return r


@jax.jit
def _run(edge_index, edge_weight):
    c = _coeffs()
    p = c @ jnp.asarray(_S_MONO)
    zeros_blk = jnp.zeros((_RPP, _N), jnp.float32)
    A_sum, A_set = _sc_scatter(edge_index, edge_weight, zeros_blk)
    colmask = jnp.any(A_set != A_sum, axis=0)
    cols = jnp.nonzero(colmask, size=_W, fill_value=0)[0]
    valid = jnp.arange(_W) < jnp.sum(colmask)
    Dc = jnp.where(valid[None, :], A_set[:, cols] - A_sum[:, cols], 0.0)
    Sel = ((cols[:, None] == jnp.arange(_N)[None, :]) & valid[:, None]).astype(jnp.float32)
    return _cheb_compute(A_sum, Dc, Sel, c, p)


def kernel(edge_index, edge_weight):
    return _run(edge_index, edge_weight)


# speedup vs baseline: 1.3702x; 1.0714x over previous
import functools

import jax
import jax.numpy as jnp
import numpy as np
from jax import lax
from jax.experimental import pallas as pl
from jax.experimental.pallas import tpu as pltpu
from jax.experimental.pallas import tpu_sc as plsc

_N = 2048
_E = 32768
_M = 8
_NPTS = _M + 1
_CB = 256
_CBP = 512
_W = 256

_NTILES = 32
_RPP = 16
_PASSES = 4
_ECHUNK = 4096
_NCHUNK = _E // _ECHUNK


def _coeffs():
    tmpN = jnp.arange(_NPTS, dtype=jnp.float32)
    num = jnp.cos(jnp.pi * (tmpN + 0.5) / _NPTS)
    kx = jnp.exp(-1.0 * (1.0 * num + 1.0)).reshape(-1, 1)
    rows = []
    for o in range(_M + 1):
        rows.append((2.0 / _NPTS) * (jnp.cos(jnp.pi * o * (tmpN + 0.5) / _NPTS).reshape(1, -1) @ kx))
    return jnp.concatenate(rows, axis=0)[:, 0]


def _s_mono_matrix():
    rows = np.zeros((9, 9))
    rows[0, 0] = 1.0
    rows[1, 1] = 0.5
    for k in range(2, 9):
        rows[k, 1:] = rows[k - 1, :8]
        rows[k] -= rows[k - 2]
    rows[0, 0] = 0.5
    return rows.astype(np.float32)


_S_MONO = _s_mono_matrix()


_UNROLL = 16


def _sc_scatter_body(key_hbm, ew_hbm, zeros_hbm,
                     asum_hbm, aset_hbm,
                     acc_sum, acc_set, kv, wv, sem0, sem1, semz):
    wid = lax.axis_index("s") * 2 + lax.axis_index("c")
    sems = [sem0, sem1]

    def start_chunk(ch, b):
        return (pltpu.async_copy(key_hbm.at[pl.ds(ch * _ECHUNK, _ECHUNK)], kv.at[b], sems[b]),
                pltpu.async_copy(ew_hbm.at[pl.ds(ch * _ECHUNK, _ECHUNK)], wv.at[b], sems[b]))

    for p in range(_PASSES):
        base = wid * (_RPP * _PASSES) + p * _RPP
        hz0 = pltpu.async_copy(zeros_hbm, acc_sum, semz)
        hz1 = pltpu.async_copy(zeros_hbm, acc_set, semz)
        pending = start_chunk(0, 0)
        hz0.wait()
        hz1.wait()
        for ch in range(_NCHUNK):
            b = ch % 2
            if ch + 1 < _NCHUNK:
                nxt = start_chunk(ch + 1, 1 - b)
            for h in pending:
                h.wait()

            lo = base * _N

            def body(k, _, lo=lo, b=b):
                for u in range(_UNROLL):
                    off = (k * _UNROLL + u) * 16
                    kvv = kv[b, pl.ds(off, 16)]
                    wvv = wv[b, pl.ds(off, 16)]
                    flat = kvv - lo
                    m = (flat >= 0) & (flat < _RPP * _N)
                    li = lax.shift_right_logical(flat, 11)
                    colv = flat & (_N - 1)
                    plsc.addupdate_scatter(acc_sum, [li, colv], wvv, mask=m)
                    plsc.store_scatter(acc_set, [li, colv], wvv, mask=m)
                return 0

            lax.fori_loop(0, _ECHUNK // (16 * _UNROLL), body, 0)
            if ch + 1 < _NCHUNK:
                pending = nxt
        pltpu.sync_copy(acc_sum, asum_hbm.at[pl.ds(base, _RPP), :])
        pltpu.sync_copy(acc_set, aset_hbm.at[pl.ds(base, _RPP), :])


def _sc_scatter(edge_index, edge_weight, zeros_blk):
    mesh = plsc.VectorSubcoreMesh(core_axis_name="c", subcore_axis_name="s")
    k = functools.partial(
        pl.kernel,
        mesh=mesh,
        compiler_params=pltpu.CompilerParams(needs_layout_passes=False),
        out_type=[jax.ShapeDtypeStruct((_N, _N), jnp.float32),
                  jax.ShapeDtypeStruct((_N, _N), jnp.float32)],
        scratch_types=[
            pltpu.VMEM((_RPP, _N), jnp.float32),
            pltpu.VMEM((_RPP, _N), jnp.float32),
            pltpu.VMEM((2, _ECHUNK), jnp.int32),
            pltpu.VMEM((2, _ECHUNK), jnp.float32),
            pltpu.SemaphoreType.DMA,
            pltpu.SemaphoreType.DMA,
            pltpu.SemaphoreType.DMA,
        ],
    )(_sc_scatter_body)
    flatkey = edge_index[0] * _N + edge_index[1]
    return k(flatkey, edge_weight, zeros_blk)


def _eye_block(j, cb=_CB):
    row = jax.lax.broadcasted_iota(jnp.int32, (_N, cb), 0)
    col = jax.lax.broadcasted_iota(jnp.int32, (_N, cb), 1) + j * cb
    return jnp.where(row == col, jnp.float32(1.0), jnp.float32(0.0))


def _pow_body(a_ref, f2_ref, f3_ref):
    j = pl.program_id(0)
    A = a_ref[...]
    Ab = a_ref[:, pl.ds(j * _CBP, _CBP)]
    eye = _eye_block(j, _CBP)
    Fb = 2.0 * Ab - 2.0 * eye
    F2b = 2.0 * jnp.dot(A, Fb, preferred_element_type=jnp.float32) - 2.0 * Fb
    F3b = 2.0 * jnp.dot(A, F2b, preferred_element_type=jnp.float32) - 2.0 * F2b
    f2_ref[...] = F2b
    f3_ref[...] = F3b


def _thin_body(a_ref, dc_ref, c_ref, corr_ref):
    A = a_ref[...]
    e1 = dc_ref[...]
    corr = e1 * c_ref[1]
    eold = jnp.zeros((_N, _W), jnp.float32)
    ecur = e1
    for k in range(2, _M + 1):
        enew = 2.0 * jnp.dot(A, ecur, preferred_element_type=jnp.float32) - 2.0 * ecur - eold
        corr = corr + enew * c_ref[k]
        eold = ecur
        ecur = enew
    corr_ref[...] = corr


def _f4_body(f2full_ref, ab_ref, f3b_ref, p_ref, f4_ref, t_ref, q0_ref):
    j = pl.program_id(0)
    F2 = f2full_ref[...]
    F2b = f2full_ref[:, pl.ds(j * _CB, _CB)]
    F4b = jnp.dot(F2, F2b, preferred_element_type=jnp.float32)
    eye = _eye_block(j)
    F3b = f3b_ref[...]
    Fb = 2.0 * ab_ref[...] - 2.0 * eye
    Tb = (p_ref[4] * eye + p_ref[5] * Fb + p_ref[6] * F2b
          + p_ref[7] * F3b + p_ref[8] * F4b)
    f4_ref[...] = F4b
    t_ref[...] = Tb
    q0_ref[...] = (p_ref[0] * eye + p_ref[1] * Fb + p_ref[2] * F2b
                   + p_ref[3] * F3b)


def _fin_body(f4full_ref, tb_ref, q0b_ref, corr_ref, selb_ref, r_ref):
    F4 = f4full_ref[...]
    Yb = jnp.dot(F4, tb_ref[...], preferred_element_type=jnp.float32)
    Cb = jnp.dot(corr_ref[...], selb_ref[...], preferred_element_type=jnp.float32)
    r_ref[...] = q0b_ref[...] + Yb + Cb


def _full_spec():
    return pl.BlockSpec((_N, _N), lambda j: (0, 0))


def _blk_spec():
    return pl.BlockSpec((_N, _CB), lambda j: (0, j))


def _smem_spec():
    return pl.BlockSpec(memory_space=pltpu.SMEM)


def _cheb_compute(A_sum, Dc, Sel, c, p):
    nblk = _N // _CB
    tc_params = pltpu.CompilerParams(vmem_limit_bytes=62 * 1024 * 1024)
    pblk = pl.BlockSpec((_N, _CBP), lambda j: (0, j))
    F2, F3 = pl.pallas_call(
        _pow_body,
        compiler_params=tc_params,
        grid=(_N // _CBP,),
        in_specs=[_full_spec()],
        out_specs=[pblk, pblk],
        out_shape=[jax.ShapeDtypeStruct((_N, _N), jnp.float32)] * 2,
    )(A_sum)
    corr = pl.pallas_call(
        _thin_body,
        compiler_params=tc_params,
        in_specs=[pl.BlockSpec((_N, _N), lambda: (0, 0)),
                  pl.BlockSpec((_N, _W), lambda: (0, 0)),
                  _smem_spec()],
        out_specs=pl.BlockSpec((_N, _W), lambda: (0, 0)),
        out_shape=jax.ShapeDtypeStruct((_N, _W), jnp.float32),
    )(A_sum, Dc, c)
    F4, T, Q0 = pl.pallas_call(
        _f4_body,
        compiler_params=tc_params,
        grid=(nblk,),
        in_specs=[_full_spec(), _blk_spec(), _blk_spec(), _smem_spec()],
        out_specs=[_blk_spec(), _blk_spec(), _blk_spec()],
        out_shape=[jax.ShapeDtypeStruct((_N, _N), jnp.float32)] * 3,
    )(F2, A_sum, F3, p)
    r = pl.pallas_call(
        _fin_body,
        compiler_params=tc_params,
        grid=(nblk,),
        in_specs=[_full_spec(), _blk_spec(), _blk_spec(),
                  pl.BlockSpec((_N, _W), lambda j: (0, 0)),
                  pl.BlockSpec((_W, _CB), lambda j: (0, j))],
        out_specs=_blk_spec(),
        out_shape=jax.ShapeDtypeStruct((_N, _N), jnp.float32),
    )(F4, T, Q0, corr, Sel)
    return r


@jax.jit
def _run(edge_index, edge_weight):
    c = _coeffs()
    p = c @ jnp.asarray(_S_MONO)
    zeros_blk = jnp.zeros((_RPP, _N), jnp.float32)
    A_sum, A_set = _sc_scatter(edge_index, edge_weight, zeros_blk)
    colmask = jnp.any(A_set != A_sum, axis=0)
    cols = jnp.nonzero(colmask, size=_W, fill_value=0)[0]
    valid = jnp.arange(_W) < jnp.sum(colmask)
    Dc = jnp.where(valid[None, :], A_set[:, cols] - A_sum[:, cols], 0.0)
    Sel = ((cols[:, None] == jnp.arange(_N)[None, :]) & valid[:, None]).astype(jnp.float32)
    return _cheb_compute(A_sum, Dc, Sel, c, p)


def kernel(edge_index, edge_weight):
    return _run(edge_index, edge_weight)
